# baseline (speedup 1.0000x reference)
import jax
import jax.numpy as jnp
from jax import lax
from jax.experimental import pallas as pl
from jax.experimental.pallas import tpu as pltpu
from jax.experimental.pallas import tpu_sc as plsc

N = 50000
E = 800000
F_IN = 128
H = 64
HH = 32
NC = 2
NS = 16
CH = 384
OUTER = 4
CPW = 136
EP = NS * CPW * CH
NOUT = CPW // OUTER
R = 2048
R4 = R // 4
R16 = R // 16
GRID = (N + R - 1) // R
NPP = GRID * R
NPP4 = NPP // 4
NPP16 = NPP // 16
RPW = NPP // NS

_f32 = jnp.float32


def _dot_t(a, w):
    return lax.dot_general(a, w, (((1,), (1,)), ((), ())),
                           preferred_element_type=_f32)


def _sc_mesh():
    return plsc.VectorSubcoreMesh(core_axis_name="c", subcore_axis_name="s",
                                  num_cores=NC, num_subcores=NS)


def _sc_count_body(dsts, z8, ones8, cnt_out, dst_v, ones_v, cacc):
    c = lax.axis_index("c")
    s = lax.axis_index("s")
    pltpu.sync_copy(z8.at[pl.ds(s * RPW, RPW)], cacc.at[pl.ds(s * RPW, RPW)])
    pltpu.sync_copy(ones8, ones_v)
    plsc.subcore_barrier()

    def outer(o, carry):
        r0 = o * OUTER
        pltpu.sync_copy(dsts.at[s, pl.ds(r0, OUTER)], dst_v)

        def inner(j, carry2):
            @pl.when((j % 2) == c)
            def _():
                pltpu.sync_copy(ones_v, cacc.at[dst_v.at[j]], add=True)

            return carry2

        return lax.fori_loop(0, OUTER, inner, carry)

    lax.fori_loop(0, NOUT, outer, 0)
    plsc.subcore_barrier()
    pltpu.sync_copy(cacc.at[pl.ds(s * RPW, RPW)],
                    cnt_out.at[c, pl.ds(s * RPW, RPW)])


def _sc_agg_body(hcat, srcs, dsts, z32, sum_out, src_v, dst_v, rows_v, acc,
                 sem0, sem1):
    c = lax.axis_index("c")
    s = lax.axis_index("s")
    pltpu.sync_copy(z32.at[pl.ds(s * RPW, RPW)], acc.at[pl.ds(s * RPW, RPW)])
    plsc.subcore_barrier()

    def outer(o, carry):
        r0 = o * OUTER
        pltpu.sync_copy(srcs.at[c, s, pl.ds(r0, OUTER)], src_v)
        pltpu.sync_copy(dsts.at[s, pl.ds(r0, OUTER)], dst_v)
        pltpu.async_copy(hcat.at[src_v.at[0]], rows_v.at[0], sem0)

        def pair(q, carry2):
            j0 = q * 2
            pltpu.make_async_copy(hcat.at[src_v.at[j0]], rows_v.at[0],
                                  sem0).wait()
            pltpu.async_copy(hcat.at[src_v.at[j0 + 1]], rows_v.at[1], sem1)
            pltpu.sync_copy(rows_v.at[0], acc.at[dst_v.at[j0]], add=True)
            pltpu.make_async_copy(hcat.at[src_v.at[j0 + 1]], rows_v.at[1],
                                  sem1).wait()

            @pl.when(j0 + 2 < OUTER)
            def _():
                pltpu.async_copy(hcat.at[src_v.at[j0 + 2]], rows_v.at[0],
                                 sem0)

            pltpu.sync_copy(rows_v.at[1], acc.at[dst_v.at[j0 + 1]], add=True)
            return carry2

        return lax.fori_loop(0, OUTER // 2, pair, carry)

    lax.fori_loop(0, NOUT, outer, 0)
    plsc.subcore_barrier()
    pltpu.sync_copy(acc.at[pl.ds(s * RPW, RPW)],
                    sum_out.at[c, pl.ds(s * RPW, RPW)])


def _sc_count(dsts, z8, ones8):
    return pl.kernel(
        _sc_count_body,
        out_type=jax.ShapeDtypeStruct((NC, NPP, 8), _f32),
        mesh=_sc_mesh(),
        compiler_params=pltpu.CompilerParams(use_tc_tiling_on_sc=False),
        scratch_types=[
            pltpu.VMEM((OUTER, CH), jnp.int32),
            pltpu.VMEM((CH, 8), _f32),
            pltpu.VMEM_SHARED((NPP, 8), _f32),
        ],
    )(dsts, z8, ones8)


def _sc_agg(hcat, srcs, dsts, z32):
    return pl.kernel(
        _sc_agg_body,
        out_type=jax.ShapeDtypeStruct((NC, NPP, HH), _f32),
        mesh=_sc_mesh(),
        compiler_params=pltpu.CompilerParams(use_tc_tiling_on_sc=False),
        scratch_types=[
            pltpu.VMEM((OUTER, CH), jnp.int32),
            pltpu.VMEM((OUTER, CH), jnp.int32),
            pltpu.VMEM((2, CH, HH), _f32),
            pltpu.VMEM_SHARED((NPP, HH), _f32),
            pltpu.SemaphoreType.DMA,
            pltpu.SemaphoreType.DMA,
        ],
    )(hcat, srcs, dsts, z32)


def _pack4(h32):
    return jnp.concatenate([h32[j * R4:(j + 1) * R4] for j in range(4)],
                           axis=-1)


def _unpack4(p128):
    return jnp.concatenate([p128[:, 32 * j:32 * (j + 1)] for j in range(4)],
                           axis=0)


def _unpack16(p128):
    return jnp.concatenate([p128[:, 8 * j:8 * (j + 1)] for j in range(16)],
                           axis=0)


def _mlp_in_body(x_ref, wp_ref, bp_ref, out_ref):
    y = _dot_t(x_ref[...], wp_ref[...])
    h = jnp.maximum(y + bp_ref[...], 0.0)
    out_ref[0] = _pack4(h[:, :HH])
    out_ref[1] = _pack4(h[:, HH:])


_mlp_in = pl.pallas_call(
    _mlp_in_body,
    grid=(GRID,),
    in_specs=[
        pl.BlockSpec((R, F_IN), lambda i: (i, 0)),
        pl.BlockSpec((H, F_IN), lambda i: (0, 0)),
        pl.BlockSpec((1, H), lambda i: (0, 0)),
    ],
    out_specs=pl.BlockSpec((NC, R4, 128), lambda i: (0, i, 0)),
    out_shape=jax.ShapeDtypeStruct((NC, NPP4, 128), _f32),
)


def _layer_math(sum_ref, cnt_ref, h_ref, wl_ref, bl_ref, wr_ref, g_ref,
                be_ref):
    sm = jnp.concatenate([_unpack4(sum_ref[0]), _unpack4(sum_ref[1])],
                         axis=-1)
    cnt = _unpack16(cnt_ref[0] + cnt_ref[1])[:, 0:1]
    mean = sm / jnp.maximum(cnt, 1.0)
    h = jnp.concatenate([_unpack4(h_ref[0]), _unpack4(h_ref[1])], axis=-1)
    y = _dot_t(mean, wl_ref[...]) + bl_ref[...] + _dot_t(h, wr_ref[...])
    mu = jnp.mean(y, axis=-1, keepdims=True)
    var = jnp.mean((y - mu) ** 2, axis=-1, keepdims=True)
    y = (y - mu) / jnp.sqrt(var + 1e-5) * g_ref[...] + be_ref[...]
    return jnp.maximum(y, 0.0) + h


def _dense_layer_body(sum_ref, cnt_ref, h_ref, wl_ref, bl_ref, wr_ref, g_ref,
                      be_ref, out_ref):
    y = _layer_math(sum_ref, cnt_ref, h_ref, wl_ref, bl_ref, wr_ref, g_ref,
                    be_ref)
    out_ref[0] = _pack4(y[:, :HH])
    out_ref[1] = _pack4(y[:, HH:])


def _dense_final_body(sum_ref, cnt_ref, h_ref, wl_ref, bl_ref, wr_ref, g_ref,
                      be_ref, w1_ref, b1_ref, w2_ref, b2_ref, out_ref):
    y = _layer_math(sum_ref, cnt_ref, h_ref, wl_ref, bl_ref, wr_ref, g_ref,
                    be_ref)
    r1 = jnp.maximum(_dot_t(y, w1_ref[...]) + b1_ref[...], 0.0)
    out_ref[...] = (jnp.sum(r1 * w2_ref[...], axis=-1, keepdims=True)
                    + b2_ref[0, 0])


def _spec_half():
    return pl.BlockSpec((NC, R4, 128), lambda i: (0, i, 0))


def _spec_cnt():
    return pl.BlockSpec((NC, R16, 128), lambda i: (0, i, 0))


def _spec_w(shape):
    nd = len(shape)
    return pl.BlockSpec(shape, (lambda i: (0, 0)) if nd == 2 else
                        (lambda i: (0,)))


_dense_layer = pl.pallas_call(
    _dense_layer_body,
    grid=(GRID,),
    in_specs=[
        _spec_half(), _spec_cnt(), _spec_half(),
        _spec_w((H, H)), _spec_w((1, H)), _spec_w((H, H)),
        _spec_w((1, H)), _spec_w((1, H)),
    ],
    out_specs=pl.BlockSpec((NC, R4, 128), lambda i: (0, i, 0)),
    out_shape=jax.ShapeDtypeStruct((NC, NPP4, 128), _f32),
)

_dense_final = pl.pallas_call(
    _dense_final_body,
    grid=(GRID,),
    in_specs=[
        _spec_half(), _spec_cnt(), _spec_half(),
        _spec_w((H, H)), _spec_w((1, H)), _spec_w((H, H)),
        _spec_w((1, H)), _spec_w((1, H)),
        _spec_w((HH, H)), _spec_w((1, HH)), _spec_w((1, HH)),
        _spec_w((1, 1)),
    ],
    out_specs=pl.BlockSpec((R, 1), lambda i: (i, 0)),
    out_shape=jax.ShapeDtypeStruct((N, 1), _f32),
)


def _perm_f(n):
    blk = (n // R) * R
    return blk + (n % R4) * 4 + (n % R) // R4


def _perm_g(n):
    blk = (n // R) * R
    return blk + (n % R16) * 16 + (n % R) // R16


def kernel(x, edge_index, Wp, bp, Wl0, bl0, Wr0, g0, be0, Wl1, bl1, Wr1, g1,
           be1, W1, b1, W2, b2):
    src = edge_index[0]
    dst = edge_index[1]
    npad = EP - E
    pad_i = jnp.arange(npad, dtype=jnp.int32)
    dummy_f = NPP - R + 3 + 4 * (pad_i % R4)
    dummy_g = NPP - R + 15 + 16 * (pad_i % R16)
    fs = jnp.concatenate([_perm_f(src), dummy_f])
    fd = jnp.concatenate([_perm_f(dst), dummy_f])
    gd = jnp.concatenate([_perm_g(dst), dummy_g])
    srcs = jnp.stack([fs, fs + NPP]).reshape(NC, NS, CPW, CH)
    dsts = fd.reshape(NS, CPW, CH)
    dstsg = gd.reshape(NS, CPW, CH)
    z32 = jnp.zeros((NPP4, 128), _f32).reshape(NPP, HH)
    z8 = jnp.zeros((NPP16, 128), _f32).reshape(NPP, 8)
    ones8 = jnp.ones((CH * 8 // 128, 128), _f32).reshape(CH, 8)

    h2p = _mlp_in(x, Wp, bp.reshape(1, H))
    cnt8 = _sc_count(dstsg, z8, ones8).reshape(NC, NPP16, 128)
    sums0 = _sc_agg(h2p.reshape(NC * NPP, HH), srcs, dsts, z32)
    h2p = _dense_layer(sums0.reshape(NC, NPP4, 128), cnt8, h2p, Wl0,
                       bl0.reshape(1, H), Wr0, g0.reshape(1, H),
                       be0.reshape(1, H))
    sums1 = _sc_agg(h2p.reshape(NC * NPP, HH), srcs, dsts, z32)
    out = _dense_final(sums1.reshape(NC, NPP4, 128), cnt8, h2p, Wl1,
                       bl1.reshape(1, H), Wr1, g1.reshape(1, H),
                       be1.reshape(1, H), W1, b1.reshape(1, HH),
                       W2.reshape(1, HH), b2.reshape(1, 1))
    return out[:, 0]


# speedup vs baseline: 1.0234x; 1.0234x over previous
import jax
import jax.numpy as jnp
from jax import lax
from jax.experimental import pallas as pl
from jax.experimental.pallas import tpu as pltpu
from jax.experimental.pallas import tpu_sc as plsc

N = 50000
E = 800000
F_IN = 128
H = 64
HH = 32
NC = 2
NS = 16
CH = 640
OUTER = 4
CPW = 80
EP = NS * CPW * CH
NOUT = CPW // OUTER
R = 2048
R4 = R // 4
R16 = R // 16
GRID = (N + R - 1) // R
NPP = GRID * R
NPP4 = NPP // 4
NPP16 = NPP // 16
RPW = NPP // NS

_f32 = jnp.float32


def _dot_t(a, w):
    return lax.dot_general(a, w, (((1,), (1,)), ((), ())),
                           preferred_element_type=_f32)


def _sc_mesh():
    return plsc.VectorSubcoreMesh(core_axis_name="c", subcore_axis_name="s",
                                  num_cores=NC, num_subcores=NS)


def _sc_count_body(dsts, z8, ones8, cnt_out, dst_v, ones_v, cacc):
    c = lax.axis_index("c")
    s = lax.axis_index("s")
    pltpu.sync_copy(z8.at[pl.ds(s * RPW, RPW)], cacc.at[pl.ds(s * RPW, RPW)])
    pltpu.sync_copy(ones8, ones_v)
    plsc.subcore_barrier()

    def outer(o, carry):
        r0 = o * OUTER
        pltpu.sync_copy(dsts.at[s, pl.ds(r0, OUTER)], dst_v)

        def inner(j, carry2):
            @pl.when((j % 2) == c)
            def _():
                pltpu.sync_copy(ones_v, cacc.at[dst_v.at[j]], add=True)

            return carry2

        return lax.fori_loop(0, OUTER, inner, carry)

    lax.fori_loop(0, NOUT, outer, 0)
    plsc.subcore_barrier()
    pltpu.sync_copy(cacc.at[pl.ds(s * RPW, RPW)],
                    cnt_out.at[c, pl.ds(s * RPW, RPW)])


def _sc_agg_body(hcat, srcs, dsts, z32, sum_out, src_v, dst_v, rows_v, acc,
                 sem0, sem1):
    c = lax.axis_index("c")
    s = lax.axis_index("s")
    pltpu.sync_copy(z32.at[pl.ds(s * RPW, RPW)], acc.at[pl.ds(s * RPW, RPW)])
    plsc.subcore_barrier()

    def outer(o, carry):
        r0 = o * OUTER
        pltpu.sync_copy(srcs.at[c, s, pl.ds(r0, OUTER)], src_v)
        pltpu.sync_copy(dsts.at[s, pl.ds(r0, OUTER)], dst_v)

        def inner(j, carry2):
            pltpu.async_copy(hcat.at[src_v.at[j]], rows_v.at[0], sem0).wait()
            pltpu.sync_copy(rows_v.at[0], acc.at[dst_v.at[j]], add=True)
            return carry2

        return lax.fori_loop(0, OUTER, inner, carry)

    lax.fori_loop(0, NOUT, outer, 0)
    plsc.subcore_barrier()
    pltpu.sync_copy(acc.at[pl.ds(s * RPW, RPW)],
                    sum_out.at[c, pl.ds(s * RPW, RPW)])


def _sc_count(dsts, z8, ones8):
    return pl.kernel(
        _sc_count_body,
        out_type=jax.ShapeDtypeStruct((NC, NPP, 8), _f32),
        mesh=_sc_mesh(),
        compiler_params=pltpu.CompilerParams(use_tc_tiling_on_sc=False),
        scratch_types=[
            pltpu.VMEM((OUTER, CH), jnp.int32),
            pltpu.VMEM((CH, 8), _f32),
            pltpu.VMEM_SHARED((NPP, 8), _f32),
        ],
    )(dsts, z8, ones8)


def _sc_agg(hcat, srcs, dsts, z32):
    return pl.kernel(
        _sc_agg_body,
        out_type=jax.ShapeDtypeStruct((NC, NPP, HH), _f32),
        mesh=_sc_mesh(),
        compiler_params=pltpu.CompilerParams(use_tc_tiling_on_sc=False),
        scratch_types=[
            pltpu.VMEM((OUTER, CH), jnp.int32),
            pltpu.VMEM((OUTER, CH), jnp.int32),
            pltpu.VMEM((1, CH, HH), _f32),
            pltpu.VMEM_SHARED((NPP, HH), _f32),
            pltpu.SemaphoreType.DMA,
            pltpu.SemaphoreType.DMA,
        ],
    )(hcat, srcs, dsts, z32)


def _pack4(h32):
    return jnp.concatenate([h32[j * R4:(j + 1) * R4] for j in range(4)],
                           axis=-1)


def _unpack4(p128):
    return jnp.concatenate([p128[:, 32 * j:32 * (j + 1)] for j in range(4)],
                           axis=0)


def _unpack16(p128):
    return jnp.concatenate([p128[:, 8 * j:8 * (j + 1)] for j in range(16)],
                           axis=0)


def _mlp_in_body(x_ref, wp_ref, bp_ref, out_ref):
    y = _dot_t(x_ref[...], wp_ref[...])
    h = jnp.maximum(y + bp_ref[...], 0.0)
    out_ref[0] = _pack4(h[:, :HH])
    out_ref[1] = _pack4(h[:, HH:])


_mlp_in = pl.pallas_call(
    _mlp_in_body,
    grid=(GRID,),
    in_specs=[
        pl.BlockSpec((R, F_IN), lambda i: (i, 0)),
        pl.BlockSpec((H, F_IN), lambda i: (0, 0)),
        pl.BlockSpec((1, H), lambda i: (0, 0)),
    ],
    out_specs=pl.BlockSpec((NC, R4, 128), lambda i: (0, i, 0)),
    out_shape=jax.ShapeDtypeStruct((NC, NPP4, 128), _f32),
)


def _layer_math(sum_ref, cnt_ref, h_ref, wl_ref, bl_ref, wr_ref, g_ref,
                be_ref):
    sm = jnp.concatenate([_unpack4(sum_ref[0]), _unpack4(sum_ref[1])],
                         axis=-1)
    cnt = _unpack16(cnt_ref[0] + cnt_ref[1])[:, 0:1]
    mean = sm / jnp.maximum(cnt, 1.0)
    h = jnp.concatenate([_unpack4(h_ref[0]), _unpack4(h_ref[1])], axis=-1)
    y = _dot_t(mean, wl_ref[...]) + bl_ref[...] + _dot_t(h, wr_ref[...])
    mu = jnp.mean(y, axis=-1, keepdims=True)
    var = jnp.mean((y - mu) ** 2, axis=-1, keepdims=True)
    y = (y - mu) / jnp.sqrt(var + 1e-5) * g_ref[...] + be_ref[...]
    return jnp.maximum(y, 0.0) + h


def _dense_layer_body(sum_ref, cnt_ref, h_ref, wl_ref, bl_ref, wr_ref, g_ref,
                      be_ref, out_ref):
    y = _layer_math(sum_ref, cnt_ref, h_ref, wl_ref, bl_ref, wr_ref, g_ref,
                    be_ref)
    out_ref[0] = _pack4(y[:, :HH])
    out_ref[1] = _pack4(y[:, HH:])


def _dense_final_body(sum_ref, cnt_ref, h_ref, wl_ref, bl_ref, wr_ref, g_ref,
                      be_ref, w1_ref, b1_ref, w2_ref, b2_ref, out_ref):
    y = _layer_math(sum_ref, cnt_ref, h_ref, wl_ref, bl_ref, wr_ref, g_ref,
                    be_ref)
    r1 = jnp.maximum(_dot_t(y, w1_ref[...]) + b1_ref[...], 0.0)
    out_ref[...] = (jnp.sum(r1 * w2_ref[...], axis=-1, keepdims=True)
                    + b2_ref[0, 0])


def _spec_half():
    return pl.BlockSpec((NC, R4, 128), lambda i: (0, i, 0))


def _spec_cnt():
    return pl.BlockSpec((NC, R16, 128), lambda i: (0, i, 0))


def _spec_w(shape):
    nd = len(shape)
    return pl.BlockSpec(shape, (lambda i: (0, 0)) if nd == 2 else
                        (lambda i: (0,)))


_dense_layer = pl.pallas_call(
    _dense_layer_body,
    grid=(GRID,),
    in_specs=[
        _spec_half(), _spec_cnt(), _spec_half(),
        _spec_w((H, H)), _spec_w((1, H)), _spec_w((H, H)),
        _spec_w((1, H)), _spec_w((1, H)),
    ],
    out_specs=pl.BlockSpec((NC, R4, 128), lambda i: (0, i, 0)),
    out_shape=jax.ShapeDtypeStruct((NC, NPP4, 128), _f32),
)

_dense_final = pl.pallas_call(
    _dense_final_body,
    grid=(GRID,),
    in_specs=[
        _spec_half(), _spec_cnt(), _spec_half(),
        _spec_w((H, H)), _spec_w((1, H)), _spec_w((H, H)),
        _spec_w((1, H)), _spec_w((1, H)),
        _spec_w((HH, H)), _spec_w((1, HH)), _spec_w((1, HH)),
        _spec_w((1, 1)),
    ],
    out_specs=pl.BlockSpec((R, 1), lambda i: (i, 0)),
    out_shape=jax.ShapeDtypeStruct((N, 1), _f32),
)


def _perm_f(n):
    blk = (n // R) * R
    return blk + (n % R4) * 4 + (n % R) // R4


def _perm_g(n):
    blk = (n // R) * R
    return blk + (n % R16) * 16 + (n % R) // R16


def kernel(x, edge_index, Wp, bp, Wl0, bl0, Wr0, g0, be0, Wl1, bl1, Wr1, g1,
           be1, W1, b1, W2, b2):
    src = edge_index[0]
    dst = edge_index[1]
    npad = EP - E
    pad_i = jnp.arange(npad, dtype=jnp.int32)
    dummy_f = NPP - R + 3 + 4 * (pad_i % R4)
    dummy_g = NPP - R + 15 + 16 * (pad_i % R16)
    fs = jnp.concatenate([_perm_f(src), dummy_f])
    fd = jnp.concatenate([_perm_f(dst), dummy_f])
    gd = jnp.concatenate([_perm_g(dst), dummy_g])
    srcs = jnp.stack([fs, fs + NPP]).reshape(NC, NS, CPW, CH)
    dsts = fd.reshape(NS, CPW, CH)
    dstsg = gd.reshape(NS, CPW, CH)
    z32 = jnp.zeros((NPP4, 128), _f32).reshape(NPP, HH)
    z8 = jnp.zeros((NPP16, 128), _f32).reshape(NPP, 8)
    ones8 = jnp.ones((CH * 8 // 128, 128), _f32).reshape(CH, 8)

    h2p = _mlp_in(x, Wp, bp.reshape(1, H))
    cnt8 = _sc_count(dstsg, z8, ones8).reshape(NC, NPP16, 128)
    sums0 = _sc_agg(h2p.reshape(NC * NPP, HH), srcs, dsts, z32)
    h2p = _dense_layer(sums0.reshape(NC, NPP4, 128), cnt8, h2p, Wl0,
                       bl0.reshape(1, H), Wr0, g0.reshape(1, H),
                       be0.reshape(1, H))
    sums1 = _sc_agg(h2p.reshape(NC * NPP, HH), srcs, dsts, z32)
    out = _dense_final(sums1.reshape(NC, NPP4, 128), cnt8, h2p, Wl1,
                       bl1.reshape(1, H), Wr1, g1.reshape(1, H),
                       be1.reshape(1, H), W1, b1.reshape(1, HH),
                       W2.reshape(1, HH), b2.reshape(1, 1))
    return out[:, 0]
